# initial kernel scaffold (unmeasured)
import jax
import jax.numpy as jnp
from jax import lax
from jax.experimental import pallas as pl
from jax.experimental.pallas import tpu as pltpu


def kernel(
    x,
):
    def body(*refs):
        pass

    out_shape = jax.ShapeDtypeStruct(..., jnp.float32)
    return pl.pallas_call(body, out_shape=out_shape)(...)


# baseline (device time: 181033 ns/iter reference)
import jax
import jax.numpy as jnp
from jax import lax
from jax.experimental import pallas as pl
from jax.experimental.pallas import tpu as pltpu

N_DEV = 8
MASKS = (1, 3, 4)


def kernel(x):
    m, n = x.shape
    assert m % (2 * N_DEV) == 0

    comm_offs = (0, m // 2, m // 2 + m // 4)
    comm_rows = m // 2 + m // 4 + m // 8

    def body(x_ref, out_ref, comm_ref, send_sems, recv_sems):
        p = lax.axis_index("i")
        b0 = jnp.bitwise_and(p, 1)
        b1 = jnp.bitwise_and(p // 2, 1)
        b2 = jnp.bitwise_and(p // 4, 1)
        keep_fns = {
            1: jnp.bitwise_xor(b0, b1) == 0,
            3: b1 == 0,
            4: b2 == 0,
        }

        barrier = pltpu.get_barrier_semaphore()
        for mask in MASKS:
            q = jnp.bitwise_xor(p, mask)
            pl.semaphore_signal(
                barrier, inc=1, device_id=(q,),
                device_id_type=pl.DeviceIdType.MESH,
            )
        pl.semaphore_wait(barrier, 3)

        out_ref[:, :] = x_ref[:, :].astype(out_ref.dtype)

        lo = jnp.int32(0)
        length = m
        for s, mask in enumerate(MASKS):
            q = jnp.bitwise_xor(p, mask)
            half = length // 2
            keep_lo = keep_fns[mask]
            send_off = pl.multiple_of(jnp.where(keep_lo, lo + half, lo), half)
            keep_off = pl.multiple_of(jnp.where(keep_lo, lo, lo + half), half)
            rdma = pltpu.make_async_remote_copy(
                src_ref=out_ref.at[pl.ds(send_off, half), :],
                dst_ref=comm_ref.at[pl.ds(comm_offs[s], half), :],
                send_sem=send_sems.at[s],
                recv_sem=recv_sems.at[s],
                device_id=(q,),
                device_id_type=pl.DeviceIdType.MESH,
            )
            rdma.start()
            rdma.wait()
            out_ref[pl.ds(keep_off, half), :] = (
                out_ref[pl.ds(keep_off, half), :]
                + comm_ref[pl.ds(comm_offs[s], half), :]
            )
            lo = keep_off
            length = half

        for s, mask in enumerate(reversed(MASKS)):
            q = jnp.bitwise_xor(p, mask)
            lo = pl.multiple_of(lo, length)
            rdma = pltpu.make_async_remote_copy(
                src_ref=out_ref.at[pl.ds(lo, length), :],
                dst_ref=out_ref.at[pl.ds(lo, length), :],
                send_sem=send_sems.at[3 + s],
                recv_sem=recv_sems.at[3 + s],
                device_id=(q,),
                device_id_type=pl.DeviceIdType.MESH,
            )
            rdma.start()
            rdma.wait()
            lo = jnp.where(keep_fns[mask], lo, lo - length)
            length = 2 * length

    return pl.pallas_call(
        body,
        out_shape=jax.ShapeDtypeStruct((m, n), jnp.bfloat16),
        in_specs=[pl.BlockSpec(memory_space=pltpu.VMEM)],
        out_specs=pl.BlockSpec(memory_space=pltpu.VMEM),
        scratch_shapes=[
            pltpu.VMEM((comm_rows, n), jnp.bfloat16),
            pltpu.SemaphoreType.DMA((6,)),
            pltpu.SemaphoreType.DMA((6,)),
        ],
        compiler_params=pltpu.CompilerParams(collective_id=0),
    )(x)


# device time: 77857 ns/iter; 2.3252x vs baseline; 2.3252x over previous
import jax
import jax.numpy as jnp
from jax import lax
from jax.experimental import pallas as pl
from jax.experimental.pallas import tpu as pltpu

N_DEV = 8

BUTTERFLIES = (
    (0, 1408, (1, 3, 4)),
    (1408, 1408, (3, 4, 1)),
    (2816, 1280, (4, 1, 3)),
)

_COMM_OFFS = {}
_off = 0
for _b, (_base, _rows, _order) in enumerate(BUTTERFLIES):
    _r = _rows
    for _s in range(3):
        _r //= 2
        _COMM_OFFS[(_b, _s)] = _off
        _off += _r
COMM_ROWS = _off


def kernel(x):
    m, n = x.shape
    assert m == sum(rows for _, rows, _ in BUTTERFLIES)

    def body(x_ref, out_ref, comm_ref, send_sems, recv_sems):
        p = lax.axis_index("i")
        b0 = jnp.bitwise_and(p, 1)
        b1 = jnp.bitwise_and(p // 2, 1)
        b2 = jnp.bitwise_and(p // 4, 1)
        keep_fns = {
            1: jnp.bitwise_xor(b0, b1) == 0,
            3: b1 == 0,
            4: b2 == 0,
        }

        barrier = pltpu.get_barrier_semaphore()
        for mask in (1, 3, 4):
            q = jnp.bitwise_xor(p, mask)
            pl.semaphore_signal(
                barrier, inc=1, device_id=(q,),
                device_id_type=pl.DeviceIdType.MESH,
            )
        pl.semaphore_wait(barrier, 3)

        out_ref[:, :] = x_ref[:, :].astype(out_ref.dtype)

        los = [jnp.int32(base) for base, _, _ in BUTTERFLIES]
        lens = [rows for _, rows, _ in BUTTERFLIES]

        for s in range(3):
            started = []
            for b, (base, rows, order) in enumerate(BUTTERFLIES):
                mask = order[s]
                q = jnp.bitwise_xor(p, mask)
                half = lens[b] // 2
                keep_lo = keep_fns[mask]
                send_off = pl.multiple_of(
                    jnp.where(keep_lo, los[b] + half, los[b]), 16
                )
                keep_off = pl.multiple_of(
                    jnp.where(keep_lo, los[b], los[b] + half), 16
                )
                rdma = pltpu.make_async_remote_copy(
                    src_ref=out_ref.at[pl.ds(send_off, half), :],
                    dst_ref=comm_ref.at[pl.ds(_COMM_OFFS[(b, s)], half), :],
                    send_sem=send_sems.at[s, b],
                    recv_sem=recv_sems.at[s, b],
                    device_id=(q,),
                    device_id_type=pl.DeviceIdType.MESH,
                )
                rdma.start()
                started.append((rdma, keep_off, half, b))
                los[b] = keep_off
                lens[b] = half
            for rdma, keep_off, half, b in started:
                rdma.wait()
                out_ref[pl.ds(keep_off, half), :] = (
                    out_ref[pl.ds(keep_off, half), :]
                    + comm_ref[pl.ds(_COMM_OFFS[(b, s)], half), :]
                )

        for s in range(3):
            started = []
            for b, (base, rows, order) in enumerate(BUTTERFLIES):
                mask = order[2 - s]
                q = jnp.bitwise_xor(p, mask)
                lo = pl.multiple_of(los[b], 16)
                rdma = pltpu.make_async_remote_copy(
                    src_ref=out_ref.at[pl.ds(lo, lens[b]), :],
                    dst_ref=out_ref.at[pl.ds(lo, lens[b]), :],
                    send_sem=send_sems.at[3 + s, b],
                    recv_sem=recv_sems.at[3 + s, b],
                    device_id=(q,),
                    device_id_type=pl.DeviceIdType.MESH,
                )
                rdma.start()
                started.append(rdma)
                los[b] = jnp.where(keep_fns[mask], los[b], los[b] - lens[b])
                lens[b] = 2 * lens[b]
            for rdma in started:
                rdma.wait()

    return pl.pallas_call(
        body,
        out_shape=jax.ShapeDtypeStruct((m, n), jnp.bfloat16),
        in_specs=[pl.BlockSpec(memory_space=pltpu.VMEM)],
        out_specs=pl.BlockSpec(memory_space=pltpu.VMEM),
        scratch_shapes=[
            pltpu.VMEM((COMM_ROWS, n), jnp.bfloat16),
            pltpu.SemaphoreType.DMA((6, 3)),
            pltpu.SemaphoreType.DMA((6, 3)),
        ],
        compiler_params=pltpu.CompilerParams(collective_id=0),
    )(x)


# device time: 75534 ns/iter; 2.3967x vs baseline; 1.0308x over previous
import jax
import jax.numpy as jnp
from jax import lax
from jax.experimental import pallas as pl
from jax.experimental.pallas import tpu as pltpu

N_DEV = 8

BUTTERFLIES = (
    (0, 1408, (1, 3, 4)),
    (1408, 1408, (3, 4, 1)),
    (2816, 1280, (4, 1, 3)),
)

_COMM_OFFS = {}
_off = 0
for _b, (_base, _rows, _order) in enumerate(BUTTERFLIES):
    _r = _rows
    for _s in range(3):
        _r //= 2
        _COMM_OFFS[(_b, _s)] = _off
        _off += _r
COMM_ROWS = _off


def kernel(x):
    m, n = x.shape
    assert m == sum(rows for _, rows, _ in BUTTERFLIES)

    def body(x_hbm, out_ref, xv_ref, comm_ref, in_sems, send_sems, recv_sems):
        p = lax.axis_index("i")
        b0 = jnp.bitwise_and(p, 1)
        b1 = jnp.bitwise_and(p // 2, 1)
        b2 = jnp.bitwise_and(p // 4, 1)
        keep_fns = {
            1: jnp.bitwise_xor(b0, b1) == 0,
            3: b1 == 0,
            4: b2 == 0,
        }

        def rs_offsets(lo, length, mask):
            half = length // 2
            keep_lo = keep_fns[mask]
            send_off = pl.multiple_of(jnp.where(keep_lo, lo + half, lo), 16)
            keep_off = pl.multiple_of(jnp.where(keep_lo, lo, lo + half), 16)
            return send_off, keep_off, half

        def start_rs(s, b, order, send_off, half):
            q = jnp.bitwise_xor(p, order[s])
            rdma = pltpu.make_async_remote_copy(
                src_ref=out_ref.at[pl.ds(send_off, half), :],
                dst_ref=comm_ref.at[pl.ds(_COMM_OFFS[(b, s)], half), :],
                send_sem=send_sems.at[s, b],
                recv_sem=recv_sems.at[s, b],
                device_id=(q,),
                device_id_type=pl.DeviceIdType.MESH,
            )
            rdma.start()
            return rdma

        def start_ag(s, b, order, lo, length):
            q = jnp.bitwise_xor(p, order[2 - s])
            rdma = pltpu.make_async_remote_copy(
                src_ref=out_ref.at[pl.ds(lo, length), :],
                dst_ref=out_ref.at[pl.ds(lo, length), :],
                send_sem=send_sems.at[3 + s, b],
                recv_sem=recv_sems.at[3 + s, b],
                device_id=(q,),
                device_id_type=pl.DeviceIdType.MESH,
            )
            rdma.start()
            return rdma

        barrier = pltpu.get_barrier_semaphore()
        for mask in (1, 3, 4):
            q = jnp.bitwise_xor(p, mask)
            pl.semaphore_signal(
                barrier, inc=1, device_id=(q,),
                device_id_type=pl.DeviceIdType.MESH,
            )
        pl.semaphore_wait(barrier, 3)

        in_dmas = []
        for b, (base, rows, order) in enumerate(BUTTERFLIES):
            cp = pltpu.make_async_copy(
                x_hbm.at[pl.ds(base, rows), :],
                xv_ref.at[pl.ds(base, rows), :],
                in_sems.at[b],
            )
            cp.start()
            in_dmas.append(cp)

        rdmas = {}
        states = []
        keep_pend = []
        for b, (base, rows, order) in enumerate(BUTTERFLIES):
            in_dmas[b].wait()
            send_off, keep_off, half = rs_offsets(jnp.int32(base), rows, order[0])
            out_ref[pl.ds(send_off, half), :] = (
                xv_ref[pl.ds(send_off, half), :].astype(out_ref.dtype)
            )
            rdmas[(0, b)] = start_rs(0, b, order, send_off, half)
            states.append((jnp.int32(base), rows))
            keep_pend.append((keep_off, half))
        for keep_off, half in keep_pend:
            out_ref[pl.ds(keep_off, half), :] = (
                xv_ref[pl.ds(keep_off, half), :].astype(out_ref.dtype)
            )

        for s in range(3):
            for b, (base, rows, order) in enumerate(BUTTERFLIES):
                lo, length = states[b]
                _, keep_off, half = rs_offsets(lo, length, order[s])
                rdmas[(s, b)].wait_recv()
                out_ref[pl.ds(keep_off, half), :] = (
                    out_ref[pl.ds(keep_off, half), :]
                    + comm_ref[pl.ds(_COMM_OFFS[(b, s)], half), :]
                )
                lo, length = keep_off, half
                states[b] = (lo, length)
                if s < 2:
                    nso, _, nh = rs_offsets(lo, length, order[s + 1])
                    rdmas[(s + 1, b)] = start_rs(s + 1, b, order, nso, nh)
                else:
                    rdmas[(3, b)] = start_ag(0, b, order, lo, length)

        for s in range(3):
            for b, (base, rows, order) in enumerate(BUTTERFLIES):
                mask = order[2 - s]
                lo, length = states[b]
                rdmas[(3 + s, b)].wait_recv()
                lo = pl.multiple_of(
                    jnp.where(keep_fns[mask], lo, lo - length), 16
                )
                length = 2 * length
                states[b] = (lo, length)
                if s < 2:
                    rdmas[(3 + s + 1, b)] = start_ag(s + 1, b, order, lo, length)

        for rdma in rdmas.values():
            rdma.wait_send()

    return pl.pallas_call(
        body,
        out_shape=jax.ShapeDtypeStruct((m, n), jnp.bfloat16),
        in_specs=[pl.BlockSpec(memory_space=pl.ANY)],
        out_specs=pl.BlockSpec(memory_space=pltpu.VMEM),
        scratch_shapes=[
            pltpu.VMEM((m, n), x.dtype),
            pltpu.VMEM((COMM_ROWS, n), jnp.bfloat16),
            pltpu.SemaphoreType.DMA((len(BUTTERFLIES),)),
            pltpu.SemaphoreType.DMA((6, 3)),
            pltpu.SemaphoreType.DMA((6, 3)),
        ],
        compiler_params=pltpu.CompilerParams(collective_id=0),
    )(x)


# device time: 69700 ns/iter; 2.5973x vs baseline; 1.0837x over previous
import jax
import jax.numpy as jnp
from jax import lax
from jax.experimental import pallas as pl
from jax.experimental.pallas import tpu as pltpu

N_DEV = 8

BUTTERFLIES = (
    (0, 1408, (1, 3, 4)),
    (1408, 1408, (3, 4, 1)),
    (2816, 1280, (4, 1, 3)),
)

_COMM_OFFS = {}
_off = 0
for _b, (_base, _rows, _order) in enumerate(BUTTERFLIES):
    _r = _rows
    for _s in range(3):
        _r //= 2
        _COMM_OFFS[(_b, _s)] = _off
        _off += _r
COMM_ROWS = _off

N_SEMS = 12


def _m8(v):
    return pl.multiple_of(v, 8)


def kernel(x):
    m, n = x.shape
    assert m == sum(rows for _, rows, _ in BUTTERFLIES)

    def body(x_hbm, out_ref, xv_ref, comm_ref, in_sems, send_sems, recv_sems):
        p = lax.axis_index("i")
        b0 = jnp.bitwise_and(p, 1)
        b1 = jnp.bitwise_and(p // 2, 1)
        b2 = jnp.bitwise_and(p // 4, 1)
        keep_fns = {
            1: jnp.bitwise_xor(b0, b1) == 0,
            3: b1 == 0,
            4: b2 == 0,
        }

        def rs_parts(s, order, lo, length):
            half = length // 2
            keep_lo = keep_fns[order[s]]
            send_off = _m8(jnp.where(keep_lo, lo + half, lo))
            keep_off = _m8(jnp.where(keep_lo, lo, lo + half))
            quarter = half // 2
            if s < 2:
                nk = keep_fns[order[s + 1]]
                crit_rel = jnp.where(nk, quarter, 0)
            else:
                crit_rel = jnp.int32(0)
            rest_rel = quarter - crit_rel
            return send_off, keep_off, half, quarter, crit_rel, rest_rel

        def start_rs_sends(s, b, order, send_off, quarter, crit_rel, rest_rel):
            qdev = jnp.bitwise_xor(p, order[s])
            out = []
            for sub, rel in ((0, crit_rel), (1, rest_rel)):
                rdma = pltpu.make_async_remote_copy(
                    src_ref=out_ref.at[pl.ds(_m8(send_off + rel), quarter), :],
                    dst_ref=comm_ref.at[
                        pl.ds(_m8(_COMM_OFFS[(b, s)] + rel), quarter), :
                    ],
                    send_sem=send_sems.at[2 * s + sub, b],
                    recv_sem=recv_sems.at[2 * s + sub, b],
                    device_id=(qdev,),
                    device_id_type=pl.DeviceIdType.MESH,
                )
                rdma.start()
                out.append(rdma)
            return out

        def start_ag_push(sem_idx, b, qdev, lo, length):
            rdma = pltpu.make_async_remote_copy(
                src_ref=out_ref.at[pl.ds(_m8(lo), length), :],
                dst_ref=out_ref.at[pl.ds(_m8(lo), length), :],
                send_sem=send_sems.at[sem_idx, b],
                recv_sem=recv_sems.at[sem_idx, b],
                device_id=(qdev,),
                device_id_type=pl.DeviceIdType.MESH,
            )
            rdma.start()
            return rdma

        def add_block(dst_off, rows_, src_off):
            out_ref[pl.ds(_m8(dst_off), rows_), :] = (
                out_ref[pl.ds(_m8(dst_off), rows_), :]
                + comm_ref[pl.ds(_m8(src_off), rows_), :]
            )

        barrier = pltpu.get_barrier_semaphore()
        for mask in (1, 3, 4):
            q = jnp.bitwise_xor(p, mask)
            pl.semaphore_signal(
                barrier, inc=1, device_id=(q,),
                device_id_type=pl.DeviceIdType.MESH,
            )
        pl.semaphore_wait(barrier, 3)

        in_dmas = []
        for b, (base, rows, order) in enumerate(BUTTERFLIES):
            cp = pltpu.make_async_copy(
                x_hbm.at[pl.ds(base, rows), :],
                xv_ref.at[pl.ds(base, rows), :],
                in_sems.at[b],
            )
            cp.start()
            in_dmas.append(cp)

        rdmas = {}
        states = []
        keep_pend = []
        for b, (base, rows, order) in enumerate(BUTTERFLIES):
            in_dmas[b].wait()
            send_off, keep_off, half, quarter, crit_rel, rest_rel = rs_parts(
                0, order, jnp.int32(base), rows
            )
            out_ref[pl.ds(send_off, half), :] = (
                xv_ref[pl.ds(send_off, half), :].astype(out_ref.dtype)
            )
            rdmas[(0, b)] = start_rs_sends(
                0, b, order, send_off, quarter, crit_rel, rest_rel
            )
            states.append((jnp.int32(base), rows))
            keep_pend.append((keep_off, half))
        for keep_off, half in keep_pend:
            out_ref[pl.ds(keep_off, half), :] = (
                xv_ref[pl.ds(keep_off, half), :].astype(out_ref.dtype)
            )

        ag_meta = []
        for s in range(3):
            for b, (base, rows, order) in enumerate(BUTTERFLIES):
                lo, length = states[b]
                send_off, keep_off, half, quarter, crit_rel, rest_rel = (
                    rs_parts(s, order, lo, length)
                )
                crit_rdma, rest_rdma = rdmas[(s, b)]
                crit_rdma.wait_recv()
                add_block(
                    keep_off + crit_rel, quarter, _COMM_OFFS[(b, s)] + crit_rel
                )
                states[b] = (keep_off, half)
                if s < 2:
                    nso, _, _, nq, ncr, nrr = rs_parts(s + 1, order, keep_off, half)
                    rdmas[(s + 1, b)] = start_rs_sends(
                        s + 1, b, order, nso, nq, ncr, nrr
                    )
                rest_rdma.wait_recv()
                add_block(
                    keep_off + rest_rel, quarter, _COMM_OFFS[(b, s)] + rest_rel
                )
                if s == 2:
                    L = half
                    lo_f = keep_off
                    q0 = jnp.bitwise_xor(p, order[2])
                    q1 = jnp.bitwise_xor(p, order[1])
                    q2 = jnp.bitwise_xor(p, order[0])
                    k0 = keep_fns[order[2]]
                    k1 = keep_fns[order[1]]
                    r0 = _m8(jnp.where(k0, lo_f + L, lo_f - L))
                    lo1 = _m8(jnp.where(k0, lo_f, lo_f - L))
                    r1 = _m8(jnp.where(k1, lo1 + 2 * L, lo1 - 2 * L))
                    rdmas[("ag_p0_q0", b)] = start_ag_push(6, b, q0, lo_f, L)
                    rdmas[("ag_p0_q1", b)] = start_ag_push(7, b, q1, lo_f, L)
                    rdmas[("ag_p0_q2", b)] = start_ag_push(9, b, q2, lo_f, L)
                    ag_meta.append((q0, q1, q2, r0, r1, L))

        for b, (q0, q1, q2, r0, r1, L) in enumerate(ag_meta):
            rdmas[("ag_p0_q0", b)].wait_recv()
            rdmas[("ag_r0_q1", b)] = start_ag_push(8, b, q1, r0, L)
            rdmas[("ag_r0_q2", b)] = start_ag_push(10, b, q2, r0, L)
        for b, (q0, q1, q2, r0, r1, L) in enumerate(ag_meta):
            rdmas[("ag_p0_q1", b)].wait_recv()
            rdmas[("ag_r0_q1", b)].wait_recv()
            rdmas[("ag_r1_q2", b)] = start_ag_push(11, b, q2, r1, 2 * L)
        for b, (q0, q1, q2, r0, r1, L) in enumerate(ag_meta):
            rdmas[("ag_p0_q2", b)].wait_recv()
            rdmas[("ag_r0_q2", b)].wait_recv()
            rdmas[("ag_r1_q2", b)].wait_recv()

        for v in rdmas.values():
            for rdma in v if isinstance(v, list) else [v]:
                rdma.wait_send()

    return pl.pallas_call(
        body,
        out_shape=jax.ShapeDtypeStruct((m, n), jnp.bfloat16),
        in_specs=[pl.BlockSpec(memory_space=pl.ANY)],
        out_specs=pl.BlockSpec(memory_space=pltpu.VMEM),
        scratch_shapes=[
            pltpu.VMEM((m, n), x.dtype),
            pltpu.VMEM((COMM_ROWS, n), jnp.bfloat16),
            pltpu.SemaphoreType.DMA((len(BUTTERFLIES),)),
            pltpu.SemaphoreType.DMA((N_SEMS, len(BUTTERFLIES))),
            pltpu.SemaphoreType.DMA((N_SEMS, len(BUTTERFLIES))),
        ],
        compiler_params=pltpu.CompilerParams(collective_id=0),
    )(x)


# device time: 69564 ns/iter; 2.6024x vs baseline; 1.0020x over previous
import jax
import jax.numpy as jnp
from jax import lax
from jax.experimental import pallas as pl
from jax.experimental.pallas import tpu as pltpu

N_DEV = 8

BUTTERFLIES = (
    (0, 1408, (1, 3, 4)),
    (1408, 1408, (3, 4, 1)),
    (2816, 1280, (4, 1, 3)),
)

_COMM_OFFS = {}
_off = 0
for _b, (_base, _rows, _order) in enumerate(BUTTERFLIES):
    _r = _rows
    for _s in range(3):
        _r //= 2
        _COMM_OFFS[(_b, _s)] = _off
        _off += _r
COMM_ROWS = _off

N_SEMS = 13


def _m8(v):
    return pl.multiple_of(v, 8)


def kernel(x):
    m, n = x.shape
    assert m == sum(rows for _, rows, _ in BUTTERFLIES)

    def body(x_hbm, out_ref, xv_ref, comm_ref, in_sems, send_sems, recv_sems):
        p = lax.axis_index("i")
        b0 = jnp.bitwise_and(p, 1)
        b1 = jnp.bitwise_and(p // 2, 1)
        b2 = jnp.bitwise_and(p // 4, 1)
        keep_fns = {
            1: jnp.bitwise_xor(b0, b1) == 0,
            3: b1 == 0,
            4: b2 == 0,
        }

        def rs_parts(s, order, lo, length):
            half = length // 2
            keep_lo = keep_fns[order[s]]
            send_off = _m8(jnp.where(keep_lo, lo + half, lo))
            keep_off = _m8(jnp.where(keep_lo, lo, lo + half))
            quarter = half // 2
            if s < 2:
                nk = keep_fns[order[s + 1]]
                crit_rel = jnp.where(nk, quarter, 0)
            else:
                crit_rel = jnp.int32(0)
            rest_rel = quarter - crit_rel
            return send_off, keep_off, half, quarter, crit_rel, rest_rel

        def start_rs_sends(s, b, order, send_off, quarter, crit_rel, rest_rel):
            qdev = jnp.bitwise_xor(p, order[s])
            out = []
            for sub, rel in ((0, crit_rel), (1, rest_rel)):
                rdma = pltpu.make_async_remote_copy(
                    src_ref=out_ref.at[pl.ds(_m8(send_off + rel), quarter), :],
                    dst_ref=comm_ref.at[
                        pl.ds(_m8(_COMM_OFFS[(b, s)] + rel), quarter), :
                    ],
                    send_sem=send_sems.at[2 * s + sub, b],
                    recv_sem=recv_sems.at[2 * s + sub, b],
                    device_id=(qdev,),
                    device_id_type=pl.DeviceIdType.MESH,
                )
                rdma.start()
                out.append(rdma)
            return out

        def start_ag_push(sem_idx, b, qdev, lo, length):
            rdma = pltpu.make_async_remote_copy(
                src_ref=out_ref.at[pl.ds(_m8(lo), length), :],
                dst_ref=out_ref.at[pl.ds(_m8(lo), length), :],
                send_sem=send_sems.at[sem_idx, b],
                recv_sem=recv_sems.at[sem_idx, b],
                device_id=(qdev,),
                device_id_type=pl.DeviceIdType.MESH,
            )
            rdma.start()
            return rdma

        def add_block(dst_off, rows_, src_off):
            out_ref[pl.ds(_m8(dst_off), rows_), :] = (
                out_ref[pl.ds(_m8(dst_off), rows_), :]
                + comm_ref[pl.ds(_m8(src_off), rows_), :]
            )

        barrier = pltpu.get_barrier_semaphore()
        for mask in (1, 3, 4):
            q = jnp.bitwise_xor(p, mask)
            pl.semaphore_signal(
                barrier, inc=1, device_id=(q,),
                device_id_type=pl.DeviceIdType.MESH,
            )
        pl.semaphore_wait(barrier, 3)

        in_dmas = []
        for b, (base, rows, order) in enumerate(BUTTERFLIES):
            cp = pltpu.make_async_copy(
                x_hbm.at[pl.ds(base, rows), :],
                xv_ref.at[pl.ds(base, rows), :],
                in_sems.at[b],
            )
            cp.start()
            in_dmas.append(cp)

        rdmas = {}
        states = []
        keep_pend = []
        for b, (base, rows, order) in enumerate(BUTTERFLIES):
            in_dmas[b].wait()
            send_off, keep_off, half, quarter, crit_rel, rest_rel = rs_parts(
                0, order, jnp.int32(base), rows
            )
            out_ref[pl.ds(send_off, half), :] = (
                xv_ref[pl.ds(send_off, half), :].astype(out_ref.dtype)
            )
            rdmas[(0, b)] = start_rs_sends(
                0, b, order, send_off, quarter, crit_rel, rest_rel
            )
            states.append((jnp.int32(base), rows))
            keep_pend.append((keep_off, half))
        for keep_off, half in keep_pend:
            out_ref[pl.ds(keep_off, half), :] = (
                xv_ref[pl.ds(keep_off, half), :].astype(out_ref.dtype)
            )

        ag_meta = []
        for s in range(3):
            for b, (base, rows, order) in enumerate(BUTTERFLIES):
                lo, length = states[b]
                send_off, keep_off, half, quarter, crit_rel, rest_rel = (
                    rs_parts(s, order, lo, length)
                )
                crit_rdma, rest_rdma = rdmas[(s, b)]
                crit_rdma.wait_recv()
                add_block(
                    keep_off + crit_rel, quarter, _COMM_OFFS[(b, s)] + crit_rel
                )
                states[b] = (keep_off, half)
                if s < 2:
                    nso, _, _, nq, ncr, nrr = rs_parts(s + 1, order, keep_off, half)
                    rdmas[(s + 1, b)] = start_rs_sends(
                        s + 1, b, order, nso, nq, ncr, nrr
                    )
                rest_rdma.wait_recv()
                add_block(
                    keep_off + rest_rel, quarter, _COMM_OFFS[(b, s)] + rest_rel
                )
                if s == 2:
                    L = half
                    lo_f = keep_off
                    rows_b = BUTTERFLIES[b][1]
                    q0 = jnp.bitwise_xor(p, order[2])
                    q1 = jnp.bitwise_xor(p, order[1])
                    q2 = jnp.bitwise_xor(p, order[0])
                    k0 = keep_fns[order[2]]
                    k1 = keep_fns[order[1]]
                    r0 = _m8(jnp.where(k0, lo_f + L, lo_f - L))
                    q1_lo_f = _m8(
                        lo_f + jnp.where(k1, rows_b // 4, -(rows_b // 4))
                    )
                    q1_r0 = _m8(jnp.where(k0, q1_lo_f + L, q1_lo_f - L))
                    rdmas[("ag_p0_q0", b)] = start_ag_push(6, b, q0, lo_f, L)
                    rdmas[("ag_p0_q1", b)] = start_ag_push(7, b, q1, lo_f, L)
                    rdmas[("ag_p0_q2", b)] = start_ag_push(9, b, q2, lo_f, L)
                    ag_meta.append((q0, q1, q2, r0, q1_lo_f, q1_r0, L))

        for b, (q0, q1, q2, r0, q1_lo_f, q1_r0, L) in enumerate(ag_meta):
            rdmas[("ag_p0_q0", b)].wait_recv()
            rdmas[("ag_r0_q1", b)] = start_ag_push(8, b, q1, r0, L)
            rdmas[("ag_r0_q2", b)] = start_ag_push(10, b, q2, r0, L)
        for b, (q0, q1, q2, r0, q1_lo_f, q1_r0, L) in enumerate(ag_meta):
            rdmas[("ag_p0_q1", b)].wait_recv()
            rdmas[("ag_q1p0_q2", b)] = start_ag_push(11, b, q2, q1_lo_f, L)
        for b, (q0, q1, q2, r0, q1_lo_f, q1_r0, L) in enumerate(ag_meta):
            rdmas[("ag_r0_q1", b)].wait_recv()
            rdmas[("ag_q1r0_q2", b)] = start_ag_push(12, b, q2, q1_r0, L)
        for b, (q0, q1, q2, r0, q1_lo_f, q1_r0, L) in enumerate(ag_meta):
            rdmas[("ag_p0_q2", b)].wait_recv()
            rdmas[("ag_r0_q2", b)].wait_recv()
            rdmas[("ag_q1p0_q2", b)].wait_recv()
            rdmas[("ag_q1r0_q2", b)].wait_recv()

        for v in rdmas.values():
            for rdma in v if isinstance(v, list) else [v]:
                rdma.wait_send()

    return pl.pallas_call(
        body,
        out_shape=jax.ShapeDtypeStruct((m, n), jnp.bfloat16),
        in_specs=[pl.BlockSpec(memory_space=pl.ANY)],
        out_specs=pl.BlockSpec(memory_space=pltpu.VMEM),
        scratch_shapes=[
            pltpu.VMEM((m, n), x.dtype),
            pltpu.VMEM((COMM_ROWS, n), jnp.bfloat16),
            pltpu.SemaphoreType.DMA((len(BUTTERFLIES),)),
            pltpu.SemaphoreType.DMA((N_SEMS, len(BUTTERFLIES))),
            pltpu.SemaphoreType.DMA((N_SEMS, len(BUTTERFLIES))),
        ],
        compiler_params=pltpu.CompilerParams(collective_id=0),
    )(x)


# device time: 69056 ns/iter; 2.6215x vs baseline; 1.0074x over previous
import jax
import jax.numpy as jnp
from jax import lax
from jax.experimental import pallas as pl
from jax.experimental.pallas import tpu as pltpu

N_DEV = 8

BUTTERFLIES = (
    (0, 1408, (1, 3, 4)),
    (1408, 1408, (3, 4, 1)),
    (2816, 1280, (4, 1, 3)),
)

_COMM_OFFS = {}
_off = 0
for _b, (_base, _rows, _order) in enumerate(BUTTERFLIES):
    _r = _rows
    for _s in range(3):
        _r //= 2
        _COMM_OFFS[(_b, _s)] = _off
        _off += _r
COMM_ROWS = _off

N_SEMS = 13


def _m8(v):
    return pl.multiple_of(v, 8)


def kernel(x):
    m, n = x.shape
    assert m == sum(rows for _, rows, _ in BUTTERFLIES)

    def body(x_hbm, out_ref, xv_ref, comm_ref, in_sems, send_sems, recv_sems):
        p = lax.axis_index("i")
        b0 = jnp.bitwise_and(p, 1)
        b1 = jnp.bitwise_and(p // 2, 1)
        b2 = jnp.bitwise_and(p // 4, 1)
        keep_fns = {
            1: jnp.bitwise_xor(b0, b1) == 0,
            3: b1 == 0,
            4: b2 == 0,
        }

        def rs_parts(s, order, lo, length):
            half = length // 2
            keep_lo = keep_fns[order[s]]
            send_off = _m8(jnp.where(keep_lo, lo + half, lo))
            keep_off = _m8(jnp.where(keep_lo, lo, lo + half))
            quarter = half // 2
            if s < 2:
                nk = keep_fns[order[s + 1]]
                crit_rel = jnp.where(nk, quarter, 0)
            else:
                crit_rel = jnp.int32(0)
            rest_rel = quarter - crit_rel
            return send_off, keep_off, half, quarter, crit_rel, rest_rel

        def start_rs_sub(s, b, order, send_off, quarter, rel, sub):
            qdev = jnp.bitwise_xor(p, order[s])
            rdma = pltpu.make_async_remote_copy(
                src_ref=out_ref.at[pl.ds(_m8(send_off + rel), quarter), :],
                dst_ref=comm_ref.at[
                    pl.ds(_m8(_COMM_OFFS[(b, s)] + rel), quarter), :
                ],
                send_sem=send_sems.at[2 * s + sub, b],
                recv_sem=recv_sems.at[2 * s + sub, b],
                device_id=(qdev,),
                device_id_type=pl.DeviceIdType.MESH,
            )
            rdma.start()
            return rdma

        def start_rs_sends(s, b, order, send_off, quarter, crit_rel, rest_rel):
            return [
                start_rs_sub(s, b, order, send_off, quarter, crit_rel, 0),
                start_rs_sub(s, b, order, send_off, quarter, rest_rel, 1),
            ]

        def start_ag_push(sem_idx, b, qdev, lo, length):
            rdma = pltpu.make_async_remote_copy(
                src_ref=out_ref.at[pl.ds(_m8(lo), length), :],
                dst_ref=out_ref.at[pl.ds(_m8(lo), length), :],
                send_sem=send_sems.at[sem_idx, b],
                recv_sem=recv_sems.at[sem_idx, b],
                device_id=(qdev,),
                device_id_type=pl.DeviceIdType.MESH,
            )
            rdma.start()
            return rdma

        def add_block(dst_off, rows_, src_off):
            out_ref[pl.ds(_m8(dst_off), rows_), :] = (
                out_ref[pl.ds(_m8(dst_off), rows_), :]
                + comm_ref[pl.ds(_m8(src_off), rows_), :]
            )

        in_dmas = []
        for b, (base, rows, order) in enumerate(BUTTERFLIES):
            half = rows // 2
            dmas = []
            for h in range(2):
                cp = pltpu.make_async_copy(
                    x_hbm.at[pl.ds(base + h * half, half), :],
                    xv_ref.at[pl.ds(base + h * half, half), :],
                    in_sems.at[b, h],
                )
                cp.start()
                dmas.append(cp)
            in_dmas.append(dmas)

        barrier = pltpu.get_barrier_semaphore()
        for mask in (1, 3, 4):
            q = jnp.bitwise_xor(p, mask)
            pl.semaphore_signal(
                barrier, inc=1, device_id=(q,),
                device_id_type=pl.DeviceIdType.MESH,
            )
        pl.semaphore_wait(barrier, 3)

        rdmas = {}
        states = []
        keep_pend = []
        for b, (base, rows, order) in enumerate(BUTTERFLIES):
            in_dmas[b][0].wait()
            in_dmas[b][1].wait()
            send_off, keep_off, half, quarter, crit_rel, rest_rel = rs_parts(
                0, order, jnp.int32(base), rows
            )
            out_ref[pl.ds(_m8(send_off + crit_rel), quarter), :] = (
                xv_ref[pl.ds(_m8(send_off + crit_rel), quarter), :]
                .astype(out_ref.dtype)
            )
            crit = start_rs_sub(0, b, order, send_off, quarter, crit_rel, 0)
            out_ref[pl.ds(_m8(send_off + rest_rel), quarter), :] = (
                xv_ref[pl.ds(_m8(send_off + rest_rel), quarter), :]
                .astype(out_ref.dtype)
            )
            rest = start_rs_sub(0, b, order, send_off, quarter, rest_rel, 1)
            rdmas[(0, b)] = [crit, rest]
            states.append((jnp.int32(base), rows))
            keep_pend.append((keep_off, half))
        for keep_off, half in keep_pend:
            out_ref[pl.ds(keep_off, half), :] = (
                xv_ref[pl.ds(keep_off, half), :].astype(out_ref.dtype)
            )

        ag_meta = []
        for s in range(3):
            for b, (base, rows, order) in enumerate(BUTTERFLIES):
                lo, length = states[b]
                send_off, keep_off, half, quarter, crit_rel, rest_rel = (
                    rs_parts(s, order, lo, length)
                )
                crit_rdma, rest_rdma = rdmas[(s, b)]
                crit_rdma.wait_recv()
                add_block(
                    keep_off + crit_rel, quarter, _COMM_OFFS[(b, s)] + crit_rel
                )
                states[b] = (keep_off, half)
                if s < 2:
                    nso, _, _, nq, ncr, nrr = rs_parts(s + 1, order, keep_off, half)
                    rdmas[(s + 1, b)] = start_rs_sends(
                        s + 1, b, order, nso, nq, ncr, nrr
                    )
                rest_rdma.wait_recv()
                add_block(
                    keep_off + rest_rel, quarter, _COMM_OFFS[(b, s)] + rest_rel
                )
                if s == 2:
                    L = half
                    lo_f = keep_off
                    rows_b = BUTTERFLIES[b][1]
                    q0 = jnp.bitwise_xor(p, order[2])
                    q1 = jnp.bitwise_xor(p, order[1])
                    q2 = jnp.bitwise_xor(p, order[0])
                    k0 = keep_fns[order[2]]
                    k1 = keep_fns[order[1]]
                    r0 = _m8(jnp.where(k0, lo_f + L, lo_f - L))
                    q1_lo_f = _m8(
                        lo_f + jnp.where(k1, rows_b // 4, -(rows_b // 4))
                    )
                    q1_r0 = _m8(jnp.where(k0, q1_lo_f + L, q1_lo_f - L))
                    rdmas[("ag_p0_q0", b)] = start_ag_push(6, b, q0, lo_f, L)
                    rdmas[("ag_p0_q1", b)] = start_ag_push(7, b, q1, lo_f, L)
                    rdmas[("ag_p0_q2", b)] = start_ag_push(9, b, q2, lo_f, L)
                    ag_meta.append((q0, q1, q2, r0, q1_lo_f, q1_r0, L))

        for b, (q0, q1, q2, r0, q1_lo_f, q1_r0, L) in enumerate(ag_meta):
            rdmas[("ag_p0_q0", b)].wait_recv()
            rdmas[("ag_r0_q1", b)] = start_ag_push(8, b, q1, r0, L)
            rdmas[("ag_r0_q2", b)] = start_ag_push(10, b, q2, r0, L)
        for b, (q0, q1, q2, r0, q1_lo_f, q1_r0, L) in enumerate(ag_meta):
            rdmas[("ag_p0_q1", b)].wait_recv()
            rdmas[("ag_q1p0_q2", b)] = start_ag_push(11, b, q2, q1_lo_f, L)
        for b, (q0, q1, q2, r0, q1_lo_f, q1_r0, L) in enumerate(ag_meta):
            rdmas[("ag_r0_q1", b)].wait_recv()
            rdmas[("ag_q1r0_q2", b)] = start_ag_push(12, b, q2, q1_r0, L)
        for b, (q0, q1, q2, r0, q1_lo_f, q1_r0, L) in enumerate(ag_meta):
            rdmas[("ag_p0_q2", b)].wait_recv()
            rdmas[("ag_r0_q2", b)].wait_recv()
            rdmas[("ag_q1p0_q2", b)].wait_recv()
            rdmas[("ag_q1r0_q2", b)].wait_recv()

        for v in rdmas.values():
            for rdma in v if isinstance(v, list) else [v]:
                rdma.wait_send()

    return pl.pallas_call(
        body,
        out_shape=jax.ShapeDtypeStruct((m, n), jnp.bfloat16),
        in_specs=[pl.BlockSpec(memory_space=pl.ANY)],
        out_specs=pl.BlockSpec(memory_space=pltpu.VMEM),
        scratch_shapes=[
            pltpu.VMEM((m, n), x.dtype),
            pltpu.VMEM((COMM_ROWS, n), jnp.bfloat16),
            pltpu.SemaphoreType.DMA((len(BUTTERFLIES), 2)),
            pltpu.SemaphoreType.DMA((N_SEMS, len(BUTTERFLIES))),
            pltpu.SemaphoreType.DMA((N_SEMS, len(BUTTERFLIES))),
        ],
        compiler_params=pltpu.CompilerParams(collective_id=0),
    )(x)


# device time: 68306 ns/iter; 2.6503x vs baseline; 1.0110x over previous
import jax
import jax.numpy as jnp
from jax import lax
from jax.experimental import pallas as pl
from jax.experimental.pallas import tpu as pltpu

N_DEV = 8

BUTTERFLIES = (
    (0, 1408, (1, 3, 4)),
    (1408, 1408, (3, 4, 1)),
    (2816, 1280, (4, 1, 3)),
)

_COMM_OFFS = {}
_off = 0
for _b, (_base, _rows, _order) in enumerate(BUTTERFLIES):
    _r = _rows
    for _s in range(3):
        _r //= 2
        _COMM_OFFS[(_b, _s)] = _off
        _off += _r
COMM_ROWS = _off

N_SEMS = 13


def _m8(v):
    return pl.multiple_of(v, 8)


def kernel(x):
    m, n = x.shape
    assert m == sum(rows for _, rows, _ in BUTTERFLIES)

    def body(x_hbm, out_ref, xv_ref, comm_ref, in_sems, send_sems, recv_sems):
        p = lax.axis_index("i")
        b0 = jnp.bitwise_and(p, 1)
        b1 = jnp.bitwise_and(p // 2, 1)
        b2 = jnp.bitwise_and(p // 4, 1)
        keep_fns = {
            1: jnp.bitwise_xor(b0, b1) == 0,
            3: b1 == 0,
            4: b2 == 0,
        }

        def rs_parts(s, order, lo, length):
            half = length // 2
            keep_lo = keep_fns[order[s]]
            send_off = _m8(jnp.where(keep_lo, lo + half, lo))
            keep_off = _m8(jnp.where(keep_lo, lo, lo + half))
            quarter = half // 2
            if s < 2:
                nk = keep_fns[order[s + 1]]
                crit_rel = jnp.where(nk, quarter, 0)
            else:
                crit_rel = jnp.int32(0)
            rest_rel = quarter - crit_rel
            return send_off, keep_off, half, quarter, crit_rel, rest_rel

        def start_rs_sub(s, b, order, send_off, quarter, rel, sub):
            qdev = jnp.bitwise_xor(p, order[s])
            rdma = pltpu.make_async_remote_copy(
                src_ref=out_ref.at[pl.ds(_m8(send_off + rel), quarter), :],
                dst_ref=comm_ref.at[
                    pl.ds(_m8(_COMM_OFFS[(b, s)] + rel), quarter), :
                ],
                send_sem=send_sems.at[2 * s + sub, b],
                recv_sem=recv_sems.at[2 * s + sub, b],
                device_id=(qdev,),
                device_id_type=pl.DeviceIdType.MESH,
            )
            rdma.start()
            return rdma

        def start_rs_sends(s, b, order, send_off, quarter, crit_rel, rest_rel):
            return [
                start_rs_sub(s, b, order, send_off, quarter, crit_rel, 0),
                start_rs_sub(s, b, order, send_off, quarter, rest_rel, 1),
            ]

        def start_ag_push(sem_idx, b, qdev, lo, length):
            rdma = pltpu.make_async_remote_copy(
                src_ref=out_ref.at[pl.ds(_m8(lo), length), :],
                dst_ref=out_ref.at[pl.ds(_m8(lo), length), :],
                send_sem=send_sems.at[sem_idx, b],
                recv_sem=recv_sems.at[sem_idx, b],
                device_id=(qdev,),
                device_id_type=pl.DeviceIdType.MESH,
            )
            rdma.start()
            return rdma

        def add_block(dst_off, rows_, src_off):
            out_ref[pl.ds(_m8(dst_off), rows_), :] = (
                out_ref[pl.ds(_m8(dst_off), rows_), :]
                + comm_ref[pl.ds(_m8(src_off), rows_), :]
            )

        in_dmas = []
        for b, (base, rows, order) in enumerate(BUTTERFLIES):
            half = rows // 2
            dmas = []
            for h in range(2):
                cp = pltpu.make_async_copy(
                    x_hbm.at[pl.ds(base + h * half, half), :],
                    xv_ref.at[pl.ds(base + h * half, half), :],
                    in_sems.at[b, h],
                )
                cp.start()
                dmas.append(cp)
            in_dmas.append(dmas)

        barrier = pltpu.get_barrier_semaphore()
        for mask in (1, 3, 4):
            q = jnp.bitwise_xor(p, mask)
            pl.semaphore_signal(
                barrier, inc=1, device_id=(q,),
                device_id_type=pl.DeviceIdType.MESH,
            )
        pl.semaphore_wait(barrier, 3)

        rdmas = {}
        states = []
        keep_pend = []
        for b, (base, rows, order) in enumerate(BUTTERFLIES):
            in_dmas[b][0].wait()
            in_dmas[b][1].wait()
            send_off, keep_off, half, quarter, crit_rel, rest_rel = rs_parts(
                0, order, jnp.int32(base), rows
            )
            out_ref[pl.ds(_m8(send_off + crit_rel), quarter), :] = (
                xv_ref[pl.ds(_m8(send_off + crit_rel), quarter), :]
                .astype(out_ref.dtype)
            )
            crit = start_rs_sub(0, b, order, send_off, quarter, crit_rel, 0)
            out_ref[pl.ds(_m8(send_off + rest_rel), quarter), :] = (
                xv_ref[pl.ds(_m8(send_off + rest_rel), quarter), :]
                .astype(out_ref.dtype)
            )
            rest = start_rs_sub(0, b, order, send_off, quarter, rest_rel, 1)
            rdmas[(0, b)] = [crit, rest]
            states.append((jnp.int32(base), rows))
            keep_pend.append((keep_off, half))
        for keep_off, half in keep_pend:
            out_ref[pl.ds(keep_off, half), :] = (
                xv_ref[pl.ds(keep_off, half), :].astype(out_ref.dtype)
            )

        ag_meta = []
        for s in range(3):
            pend = []
            for b, (base, rows, order) in enumerate(BUTTERFLIES):
                lo, length = states[b]
                send_off, keep_off, half, quarter, crit_rel, rest_rel = (
                    rs_parts(s, order, lo, length)
                )
                crit_rdma, rest_rdma = rdmas[(s, b)]
                crit_rdma.wait_recv()
                add_block(
                    keep_off + crit_rel, quarter, _COMM_OFFS[(b, s)] + crit_rel
                )
                states[b] = (keep_off, half)
                if s < 2:
                    nso, _, _, nq, ncr, nrr = rs_parts(s + 1, order, keep_off, half)
                    rdmas[(s + 1, b)] = start_rs_sends(
                        s + 1, b, order, nso, nq, ncr, nrr
                    )
                pend.append((rest_rdma, keep_off, half, quarter, rest_rel))
            for b, (rest_rdma, keep_off, half, quarter, rest_rel) in enumerate(pend):
                order = BUTTERFLIES[b][2]
                rest_rdma.wait_recv()
                add_block(
                    keep_off + rest_rel, quarter, _COMM_OFFS[(b, s)] + rest_rel
                )
                if s == 2:
                    L = half
                    lo_f = keep_off
                    rows_b = BUTTERFLIES[b][1]
                    q0 = jnp.bitwise_xor(p, order[2])
                    q1 = jnp.bitwise_xor(p, order[1])
                    q2 = jnp.bitwise_xor(p, order[0])
                    k0 = keep_fns[order[2]]
                    k1 = keep_fns[order[1]]
                    r0 = _m8(jnp.where(k0, lo_f + L, lo_f - L))
                    q1_lo_f = _m8(
                        lo_f + jnp.where(k1, rows_b // 4, -(rows_b // 4))
                    )
                    q1_r0 = _m8(jnp.where(k0, q1_lo_f + L, q1_lo_f - L))
                    rdmas[("ag_p0_q0", b)] = start_ag_push(6, b, q0, lo_f, L)
                    rdmas[("ag_p0_q1", b)] = start_ag_push(7, b, q1, lo_f, L)
                    rdmas[("ag_p0_q2", b)] = start_ag_push(9, b, q2, lo_f, L)
                    ag_meta.append((q0, q1, q2, r0, q1_lo_f, q1_r0, L))

        for b, (q0, q1, q2, r0, q1_lo_f, q1_r0, L) in enumerate(ag_meta):
            rdmas[("ag_p0_q0", b)].wait_recv()
            rdmas[("ag_r0_q1", b)] = start_ag_push(8, b, q1, r0, L)
            rdmas[("ag_r0_q2", b)] = start_ag_push(10, b, q2, r0, L)
        for b, (q0, q1, q2, r0, q1_lo_f, q1_r0, L) in enumerate(ag_meta):
            rdmas[("ag_p0_q1", b)].wait_recv()
            rdmas[("ag_q1p0_q2", b)] = start_ag_push(11, b, q2, q1_lo_f, L)
        for b, (q0, q1, q2, r0, q1_lo_f, q1_r0, L) in enumerate(ag_meta):
            rdmas[("ag_r0_q1", b)].wait_recv()
            rdmas[("ag_q1r0_q2", b)] = start_ag_push(12, b, q2, q1_r0, L)
        for b, (q0, q1, q2, r0, q1_lo_f, q1_r0, L) in enumerate(ag_meta):
            rdmas[("ag_p0_q2", b)].wait_recv()
            rdmas[("ag_r0_q2", b)].wait_recv()
            rdmas[("ag_q1p0_q2", b)].wait_recv()
            rdmas[("ag_q1r0_q2", b)].wait_recv()

        for v in rdmas.values():
            for rdma in v if isinstance(v, list) else [v]:
                rdma.wait_send()

    return pl.pallas_call(
        body,
        out_shape=jax.ShapeDtypeStruct((m, n), jnp.bfloat16),
        in_specs=[pl.BlockSpec(memory_space=pl.ANY)],
        out_specs=pl.BlockSpec(memory_space=pltpu.VMEM),
        scratch_shapes=[
            pltpu.VMEM((m, n), x.dtype),
            pltpu.VMEM((COMM_ROWS, n), jnp.bfloat16),
            pltpu.SemaphoreType.DMA((len(BUTTERFLIES), 2)),
            pltpu.SemaphoreType.DMA((N_SEMS, len(BUTTERFLIES))),
            pltpu.SemaphoreType.DMA((N_SEMS, len(BUTTERFLIES))),
        ],
        compiler_params=pltpu.CompilerParams(collective_id=0),
    )(x)
